# baseline (speedup 1.0000x reference)
import functools
import jax
import jax.numpy as jnp
from jax import lax
from jax.experimental import pallas as pl
from jax.experimental.pallas import tpu as pltpu
from jax.experimental.pallas import tpu_sc as plsc

NC = 2
NS = 16
L = 16
NW = NC * NS
CH = 64
AC = 128


def _make_sc_aggregate(n_nodes, n_edges, dim):
    assert n_edges % AC == 0
    nchunks_ac = n_edges // AC
    mpt = nchunks_ac // NW
    mlen = mpt * AC
    ntail = nchunks_ac - mpt * NW
    assert ntail <= NW
    assert mlen % CH == 0
    nch = mlen // CH
    assert nch % 2 == 0
    nzfull = n_nodes // CH
    nzrem = n_nodes - nzfull * CH
    assert nzrem % 8 == 0

    mesh = plsc.VectorSubcoreMesh(
        core_axis_name="c", subcore_axis_name="s",
        num_cores=NC, num_subcores=NS)

    @functools.partial(
        pl.kernel,
        out_type=[
            jax.ShapeDtypeStruct((NC, n_nodes, dim), jnp.float32),
            jax.ShapeDtypeStruct((NW * n_nodes,), jnp.float32),
        ],
        mesh=mesh,
        compiler_params=pltpu.CompilerParams(needs_layout_passes=False),
        scratch_types=[
            pltpu.VMEM((2, mlen), jnp.int32),
            pltpu.VMEM((2, AC), jnp.int32),
            pltpu.VMEM((CH,), jnp.int32),
            pltpu.VMEM((CH,), jnp.int32),
            pltpu.VMEM((CH, dim), jnp.float32),
            pltpu.VMEM((CH, dim), jnp.float32),
            pltpu.VMEM((n_nodes,), jnp.float32),
            pltpu.VMEM_SHARED((n_nodes, dim), jnp.float32),
            pltpu.SemaphoreType.DMA,
            pltpu.SemaphoreType.DMA,
        ],
    )
    def sc_agg(ei_hbm, emb_hbm, agg_out, deg_out,
               ei_v, tail_v, dst_idx_a, dst_idx_b, rows_a, rows_b,
               deg_v, agg_sh, sem_a, sem_b):
        c = lax.axis_index("c")
        s = lax.axis_index("s")
        wid = c * NS + s
        base = wid * mlen

        pltpu.async_copy(ei_hbm.at[:, pl.ds(base, mlen)], ei_v, sem_a)

        zeros16 = jnp.zeros((L,), jnp.float32)

        def zb_body(i, _):
            rows_a[i // (dim // L), pl.ds((i % (dim // L)) * L, L)] = zeros16
            return 0
        lax.fori_loop(0, CH * (dim // L), zb_body, 0, unroll=8)

        def zd_body(i, _):
            deg_v[pl.ds(i * L, L)] = zeros16
            return 0
        lax.fori_loop(0, n_nodes // L, zd_body, 0, unroll=8)

        def zs_body(k, _):
            idx = k * NS + s
            @pl.when(idx < nzfull)
            def _():
                pltpu.sync_copy(rows_a, agg_sh.at[pl.ds(idx * CH, CH)])
            if nzrem:
                @pl.when(idx == nzfull)
                def _():
                    pltpu.sync_copy(rows_a.at[pl.ds(0, nzrem)],
                                    agg_sh.at[pl.ds(nzfull * CH, nzrem)])
            return 0
        lax.fori_loop(0, pl.cdiv(nzfull + 1, NS), zs_body, 0)

        pltpu.make_async_copy(ei_hbm.at[:, pl.ds(base, mlen)], ei_v, sem_a).wait()

        plsc.subcore_barrier()

        ones16 = jnp.full((L,), 1.0, jnp.float32)

        def stage(j, dst_idx):
            for i in range(CH // L):
                dst_idx[pl.ds(i * L, L)] = ei_v[1, pl.ds(j * CH + i * L, L)]

        def gather_start(j, rows, sem):
            pltpu.async_copy(emb_hbm.at[ei_v.at[0, pl.ds(j * CH, CH)]], rows, sem)

        def gather_wait(j, rows, sem):
            pltpu.make_async_copy(
                emb_hbm.at[ei_v.at[0, pl.ds(j * CH, CH)]], rows, sem).wait()

        def scatter_deg(rows, dst_idx):
            pltpu.sync_copy(rows, agg_sh.at[dst_idx], add=True)
            for i in range(CH // L):
                plsc.addupdate_scatter(deg_v, [dst_idx[pl.ds(i * L, L)]], ones16)

        stage(0, dst_idx_a)
        gather_start(0, rows_a, sem_a)

        def chunk_body(jj, _):
            j0 = jj * 2
            stage(j0 + 1, dst_idx_b)
            gather_start(j0 + 1, rows_b, sem_b)
            gather_wait(j0, rows_a, sem_a)
            scatter_deg(rows_a, dst_idx_a)

            @pl.when(j0 + 2 < nch)
            def _():
                stage(j0 + 2, dst_idx_a)
                gather_start(j0 + 2, rows_a, sem_a)
            gather_wait(j0 + 1, rows_b, sem_b)
            scatter_deg(rows_b, dst_idx_b)
            return 0
        lax.fori_loop(0, nch // 2, chunk_body, 0)

        if ntail:
            @pl.when(wid < ntail)
            def _():
                tbase = (mpt * NW + wid) * AC
                pltpu.sync_copy(ei_hbm.at[:, pl.ds(tbase, AC)], tail_v)
                for h in range(AC // CH):
                    pltpu.async_copy(
                        emb_hbm.at[tail_v.at[0, pl.ds(h * CH, CH)]],
                        rows_a, sem_a)
                    for i in range(CH // L):
                        dst_idx_a[pl.ds(i * L, L)] = (
                            tail_v[1, pl.ds(h * CH + i * L, L)])
                    pltpu.make_async_copy(
                        emb_hbm.at[tail_v.at[0, pl.ds(h * CH, CH)]],
                        rows_a, sem_a).wait()
                    scatter_deg(rows_a, dst_idx_a)

        plsc.subcore_barrier()

        pltpu.async_copy(deg_v, deg_out.at[pl.ds(wid * n_nodes, n_nodes)], sem_a)

        def co_body(k, _):
            idx = k * NS + s
            @pl.when(idx < nzfull)
            def _():
                pltpu.sync_copy(agg_sh.at[pl.ds(idx * CH, CH)],
                                agg_out.at[c, pl.ds(idx * CH, CH)])
            if nzrem:
                @pl.when(idx == nzfull)
                def _():
                    pltpu.sync_copy(agg_sh.at[pl.ds(nzfull * CH, nzrem)],
                                    agg_out.at[c, pl.ds(nzfull * CH, nzrem)])
            return 0
        lax.fori_loop(0, pl.cdiv(nzfull + 1, NS), co_body, 0)
        pltpu.make_async_copy(deg_v, deg_out.at[pl.ds(wid * n_nodes, n_nodes)],
                              sem_a).wait()

    return sc_agg


def _make_tc_right(n_nodes, dim, hdim, blk):
    nblk = n_nodes // blk

    def tc_body(emb_ref, wr_ref, bl_ref, out_ref):
        out_ref[...] = (
            jnp.dot(emb_ref[...], wr_ref[...], preferred_element_type=jnp.float32)
            + bl_ref[...]
        )

    return pl.pallas_call(
        tc_body,
        grid=(nblk,),
        in_specs=[
            pl.BlockSpec((blk, dim), lambda i: (i, 0)),
            pl.BlockSpec((dim, hdim), lambda i: (0, 0)),
            pl.BlockSpec((1, hdim), lambda i: (0, 0)),
        ],
        out_specs=pl.BlockSpec((blk, hdim), lambda i: (i, 0)),
        out_shape=jax.ShapeDtypeStruct((n_nodes, hdim), jnp.float32),
    )


def _make_tc_combine(n_nodes, dim, hdim):
    def tc_body(agg_ref, deg_ref, outr_ref, wl_ref, out_ref):
        agg = agg_ref[0] + agg_ref[1]
        deg = jnp.sum(deg_ref[...], axis=0)
        deg = jnp.maximum(deg, 1.0)
        mean = agg * (1.0 / deg)[:, None]
        out_ref[...] = (
            jnp.dot(mean, wl_ref[...], preferred_element_type=jnp.float32)
            + outr_ref[...]
        )

    return pl.pallas_call(
        tc_body,
        out_shape=jax.ShapeDtypeStruct((n_nodes, hdim), jnp.float32),
    )


def kernel(x, edge_index, emb_weight, W_l, b_l, W_r):
    del x
    n_nodes, dim = emb_weight.shape
    n_edges = edge_index.shape[1]
    hdim = W_l.shape[1]

    sc_agg = _make_sc_aggregate(n_nodes, n_edges, dim)
    agg_p, deg_p = sc_agg(edge_index, emb_weight)
    deg2 = deg_p.reshape(NW, n_nodes)

    out_r = _make_tc_right(n_nodes, dim, hdim, blk=2000)(
        emb_weight, W_r, b_l.reshape(1, hdim))
    tc_combine = _make_tc_combine(n_nodes, dim, hdim)
    return tc_combine(agg_p, deg2, out_r, W_l)


# speedup vs baseline: 1.0243x; 1.0243x over previous
import functools
import jax
import jax.numpy as jnp
from jax import lax
from jax.experimental import pallas as pl
from jax.experimental.pallas import tpu as pltpu
from jax.experimental.pallas import tpu_sc as plsc

NC = 2
NS = 16
L = 16
NW = NC * NS
CH = 128


def _make_sc_aggregate(n_nodes, n_edges, dim):
    assert n_edges % CH == 0
    nchunks = n_edges // CH
    mpt = nchunks // NW
    ntail = nchunks - mpt * NW
    assert ntail <= NW
    nch = mpt
    assert nch % 2 == 0
    nzfull = n_nodes // CH
    nzrem = n_nodes - nzfull * CH
    assert nzrem % 8 == 0

    mesh = plsc.VectorSubcoreMesh(
        core_axis_name="c", subcore_axis_name="s",
        num_cores=NC, num_subcores=NS)

    @functools.partial(
        pl.kernel,
        out_type=[
            jax.ShapeDtypeStruct((NC, n_nodes, dim), jnp.float32),
            jax.ShapeDtypeStruct((NW * n_nodes,), jnp.float32),
        ],
        mesh=mesh,
        compiler_params=pltpu.CompilerParams(needs_layout_passes=False),
        scratch_types=[
            pltpu.VMEM((2, CH), jnp.int32),
            pltpu.VMEM((2, CH), jnp.int32),
            pltpu.VMEM((CH, dim), jnp.float32),
            pltpu.VMEM((CH, dim), jnp.float32),
            pltpu.VMEM((n_nodes,), jnp.float32),
            pltpu.VMEM_SHARED((n_nodes, dim), jnp.float32),
            pltpu.SemaphoreType.DMA,
            pltpu.SemaphoreType.DMA,
            pltpu.SemaphoreType.DMA,
            pltpu.SemaphoreType.DMA,
        ],
    )
    def sc_agg(ei_hbm, emb_hbm, agg_out, deg_out,
               idx_a, idx_b, rows_a, rows_b, deg_v, agg_sh,
               gsa, gsb, isa, isb):
        c = lax.axis_index("c")
        s = lax.axis_index("s")
        wid = c * NS + s
        cbase = wid * mpt

        def idx_start(j, ibuf, isem):
            pltpu.async_copy(
                ei_hbm.at[:, pl.ds((cbase + j) * CH, CH)], ibuf, isem)

        def idx_wait(j, ibuf, isem):
            pltpu.make_async_copy(
                ei_hbm.at[:, pl.ds((cbase + j) * CH, CH)], ibuf, isem).wait()

        def gather_start(ibuf, rows, gsem):
            pltpu.async_copy(emb_hbm.at[ibuf.at[0]], rows, gsem)

        def gather_wait(ibuf, rows, gsem):
            pltpu.make_async_copy(emb_hbm.at[ibuf.at[0]], rows, gsem).wait()

        idx_start(0, idx_a, isa)
        idx_start(1, idx_b, isb)

        zeros16 = jnp.zeros((L,), jnp.float32)

        def zb_body(i, _):
            rows_a[i // (dim // L), pl.ds((i % (dim // L)) * L, L)] = zeros16
            return 0
        lax.fori_loop(0, CH * (dim // L), zb_body, 0, unroll=8)

        def zd_body(i, _):
            deg_v[pl.ds(i * L, L)] = zeros16
            return 0
        lax.fori_loop(0, n_nodes // L, zd_body, 0, unroll=8)

        def zs_body(k, _):
            idx = k * NS + s
            @pl.when(idx < nzfull)
            def _():
                pltpu.sync_copy(rows_a, agg_sh.at[pl.ds(idx * CH, CH)])
            if nzrem:
                @pl.when(idx == nzfull)
                def _():
                    pltpu.sync_copy(rows_a.at[pl.ds(0, nzrem)],
                                    agg_sh.at[pl.ds(nzfull * CH, nzrem)])
            return 0
        lax.fori_loop(0, pl.cdiv(nzfull + 1, NS), zs_body, 0)

        idx_wait(0, idx_a, isa)
        plsc.subcore_barrier()

        ones16 = jnp.full((L,), 1.0, jnp.float32)

        def scatter_deg(rows, ibuf):
            pltpu.sync_copy(rows, agg_sh.at[ibuf.at[1]], add=True)
            for i in range(CH // L):
                plsc.addupdate_scatter(deg_v, [ibuf[1, pl.ds(i * L, L)]], ones16)

        gather_start(idx_a, rows_a, gsa)

        def chunk_body(jj, _):
            j0 = jj * 2
            gather_wait(idx_a, rows_a, gsa)
            idx_wait(j0 + 1, idx_b, isb)
            gather_start(idx_b, rows_b, gsb)
            scatter_deg(rows_a, idx_a)

            @pl.when(j0 + 2 < nch)
            def _():
                idx_start(j0 + 2, idx_a, isa)
            gather_wait(idx_b, rows_b, gsb)

            @pl.when(j0 + 2 < nch)
            def _():
                idx_wait(j0 + 2, idx_a, isa)
                gather_start(idx_a, rows_a, gsa)
            scatter_deg(rows_b, idx_b)

            @pl.when(j0 + 3 < nch)
            def _():
                idx_start(j0 + 3, idx_b, isb)
            return 0
        lax.fori_loop(0, nch // 2, chunk_body, 0)

        if ntail:
            @pl.when(wid < ntail)
            def _():
                tbase = (mpt * NW + wid) * CH
                pltpu.sync_copy(ei_hbm.at[:, pl.ds(tbase, CH)], idx_a)
                pltpu.async_copy(emb_hbm.at[idx_a.at[0]], rows_a, gsa)
                pltpu.make_async_copy(emb_hbm.at[idx_a.at[0]], rows_a, gsa).wait()
                scatter_deg(rows_a, idx_a)

        plsc.subcore_barrier()

        pltpu.async_copy(deg_v, deg_out.at[pl.ds(wid * n_nodes, n_nodes)], isa)

        def co_body(k, _):
            idx = k * NS + s
            @pl.when(idx < nzfull)
            def _():
                pltpu.sync_copy(agg_sh.at[pl.ds(idx * CH, CH)],
                                agg_out.at[c, pl.ds(idx * CH, CH)])
            if nzrem:
                @pl.when(idx == nzfull)
                def _():
                    pltpu.sync_copy(agg_sh.at[pl.ds(nzfull * CH, nzrem)],
                                    agg_out.at[c, pl.ds(nzfull * CH, nzrem)])
            return 0
        lax.fori_loop(0, pl.cdiv(nzfull + 1, NS), co_body, 0)
        pltpu.make_async_copy(deg_v, deg_out.at[pl.ds(wid * n_nodes, n_nodes)],
                              isa).wait()

    return sc_agg


def _make_tc_right(n_nodes, dim, hdim, blk):
    nblk = n_nodes // blk

    def tc_body(emb_ref, wr_ref, bl_ref, out_ref):
        out_ref[...] = (
            jnp.dot(emb_ref[...], wr_ref[...], preferred_element_type=jnp.float32)
            + bl_ref[...]
        )

    return pl.pallas_call(
        tc_body,
        grid=(nblk,),
        in_specs=[
            pl.BlockSpec((blk, dim), lambda i: (i, 0)),
            pl.BlockSpec((dim, hdim), lambda i: (0, 0)),
            pl.BlockSpec((1, hdim), lambda i: (0, 0)),
        ],
        out_specs=pl.BlockSpec((blk, hdim), lambda i: (i, 0)),
        out_shape=jax.ShapeDtypeStruct((n_nodes, hdim), jnp.float32),
    )


def _make_tc_combine(n_nodes, dim, hdim):
    def tc_body(agg_ref, deg_ref, outr_ref, wl_ref, out_ref):
        agg = agg_ref[0] + agg_ref[1]
        deg = jnp.sum(deg_ref[...], axis=0)
        deg = jnp.maximum(deg, 1.0)
        mean = agg * (1.0 / deg)[:, None]
        out_ref[...] = (
            jnp.dot(mean, wl_ref[...], preferred_element_type=jnp.float32)
            + outr_ref[...]
        )

    return pl.pallas_call(
        tc_body,
        out_shape=jax.ShapeDtypeStruct((n_nodes, hdim), jnp.float32),
    )


def kernel(x, edge_index, emb_weight, W_l, b_l, W_r):
    del x
    n_nodes, dim = emb_weight.shape
    n_edges = edge_index.shape[1]
    hdim = W_l.shape[1]

    sc_agg = _make_sc_aggregate(n_nodes, n_edges, dim)
    agg_p, deg_p = sc_agg(edge_index, emb_weight)
    deg2 = deg_p.reshape(NW, n_nodes)

    out_r = _make_tc_right(n_nodes, dim, hdim, blk=2000)(
        emb_weight, W_r, b_l.reshape(1, hdim))
    tc_combine = _make_tc_combine(n_nodes, dim, hdim)
    return tc_combine(agg_p, deg2, out_r, W_l)


# speedup vs baseline: 1.0838x; 1.0581x over previous
import functools
import jax
import jax.numpy as jnp
from jax import lax
from jax.experimental import pallas as pl
from jax.experimental.pallas import tpu as pltpu
from jax.experimental.pallas import tpu_sc as plsc

NC = 2
NS = 16
L = 16
NW = NC * NS
CH = 80


def _make_sc_aggregate(n_nodes, n_edges, dim):
    assert n_edges % NW == 0
    ept = n_edges // NW
    assert ept % CH == 0
    nch = ept // CH
    assert n_nodes % CH == 0
    nzch = n_nodes // CH

    mesh = plsc.VectorSubcoreMesh(
        core_axis_name="c", subcore_axis_name="s",
        num_cores=NC, num_subcores=NS)

    @functools.partial(
        pl.kernel,
        out_type=[
            jax.ShapeDtypeStruct((NC, n_nodes, dim), jnp.float32),
            jax.ShapeDtypeStruct((NW * n_nodes,), jnp.float32),
        ],
        mesh=mesh,
        compiler_params=pltpu.CompilerParams(needs_layout_passes=False),
        scratch_types=[
            pltpu.VMEM((ept,), jnp.int32),
            pltpu.VMEM((ept,), jnp.int32),
            pltpu.VMEM((CH,), jnp.int32),
            pltpu.VMEM((CH,), jnp.int32),
            pltpu.VMEM((CH, dim), jnp.float32),
            pltpu.VMEM((CH, dim), jnp.float32),
            pltpu.VMEM((n_nodes,), jnp.float32),
            pltpu.VMEM_SHARED((n_nodes, dim), jnp.float32),
            pltpu.SemaphoreType.DMA,
            pltpu.SemaphoreType.DMA,
        ],
    )
    def sc_agg(ei_hbm, emb_hbm, agg_out, deg_out,
               src_v, dst_v, dst_idx_a, dst_idx_b, rows_a, rows_b,
               deg_v, agg_sh, sem_a, sem_b):
        c = lax.axis_index("c")
        s = lax.axis_index("s")
        wid = c * NS + s
        base = wid * ept

        pltpu.async_copy(ei_hbm.at[pl.ds(base, ept)], src_v, sem_a)
        pltpu.async_copy(ei_hbm.at[pl.ds(n_edges + base, ept)], dst_v, sem_b)

        zeros16 = jnp.zeros((L,), jnp.float32)

        def zb_body(i, _):
            rows_a[i // (dim // L), pl.ds((i % (dim // L)) * L, L)] = zeros16
            return 0
        lax.fori_loop(0, CH * (dim // L), zb_body, 0, unroll=8)

        def zd_body(i, _):
            deg_v[pl.ds(i * L, L)] = zeros16
            return 0
        lax.fori_loop(0, n_nodes // L, zd_body, 0, unroll=8)

        def zs_body(k, _):
            @pl.when(k * NS + s < nzch)
            def _():
                r0 = (k * NS + s) * CH
                pltpu.sync_copy(rows_a, agg_sh.at[pl.ds(r0, CH)])
            return 0
        lax.fori_loop(0, pl.cdiv(nzch, NS), zs_body, 0)

        pltpu.make_async_copy(ei_hbm.at[pl.ds(base, ept)], src_v, sem_a).wait()
        pltpu.make_async_copy(ei_hbm.at[pl.ds(n_edges + base, ept)], dst_v,
                              sem_b).wait()

        plsc.subcore_barrier()

        ones16 = jnp.full((L,), 1.0, jnp.float32)

        def stage(j, dst_idx):
            for i in range(CH // L):
                dst_idx[pl.ds(i * L, L)] = dst_v[pl.ds(j * CH + i * L, L)]

        def gather_start(j, rows):
            return pltpu.async_copy(
                emb_hbm.at[src_v.at[pl.ds(j * CH, CH)]], rows,
                sem_a if rows is rows_a else sem_b)

        def gather_wait(j, rows):
            pltpu.make_async_copy(
                emb_hbm.at[src_v.at[pl.ds(j * CH, CH)]], rows,
                sem_a if rows is rows_a else sem_b).wait()

        def consume(j, rows, dst_idx):
            gather_wait(j, rows)
            pltpu.sync_copy(rows, agg_sh.at[dst_idx], add=True)
            for i in range(CH // L):
                plsc.addupdate_scatter(deg_v, [dst_idx[pl.ds(i * L, L)]], ones16)

        assert nch % 2 == 1
        stage(0, dst_idx_a)
        gather_start(0, rows_a)

        def chunk_body(jj, _):
            j0 = jj * 2
            stage(j0 + 1, dst_idx_b)
            gather_start(j0 + 1, rows_b)
            consume(j0, rows_a, dst_idx_a)
            stage(j0 + 2, dst_idx_a)
            gather_start(j0 + 2, rows_a)
            consume(j0 + 1, rows_b, dst_idx_b)
            return 0
        lax.fori_loop(0, (nch - 1) // 2, chunk_body, 0)
        consume(nch - 1, rows_a, dst_idx_a)

        plsc.subcore_barrier()

        pltpu.async_copy(deg_v, deg_out.at[pl.ds(wid * n_nodes, n_nodes)], sem_a)

        def co_body(k, _):
            @pl.when(k * NS + s < nzch)
            def _():
                r0 = (k * NS + s) * CH
                pltpu.sync_copy(agg_sh.at[pl.ds(r0, CH)],
                                agg_out.at[c, pl.ds(r0, CH)])
            return 0
        lax.fori_loop(0, pl.cdiv(nzch, NS), co_body, 0)
        pltpu.make_async_copy(deg_v, deg_out.at[pl.ds(wid * n_nodes, n_nodes)],
                              sem_a).wait()

    return sc_agg


def _make_tc_right(n_nodes, dim, hdim, blk):
    nblk = n_nodes // blk

    def tc_body(emb_ref, wr_ref, bl_ref, out_ref):
        out_ref[...] = (
            jnp.dot(emb_ref[...], wr_ref[...], preferred_element_type=jnp.float32)
            + bl_ref[...]
        )

    return pl.pallas_call(
        tc_body,
        grid=(nblk,),
        in_specs=[
            pl.BlockSpec((blk, dim), lambda i: (i, 0)),
            pl.BlockSpec((dim, hdim), lambda i: (0, 0)),
            pl.BlockSpec((1, hdim), lambda i: (0, 0)),
        ],
        out_specs=pl.BlockSpec((blk, hdim), lambda i: (i, 0)),
        out_shape=jax.ShapeDtypeStruct((n_nodes, hdim), jnp.float32),
    )


def _make_tc_combine(n_nodes, dim, hdim):
    def tc_body(agg_ref, deg_ref, outr_ref, wl_ref, out_ref):
        agg = agg_ref[0] + agg_ref[1]
        deg = jnp.sum(deg_ref[...], axis=0)
        deg = jnp.maximum(deg, 1.0)
        mean = agg * (1.0 / deg)[:, None]
        out_ref[...] = (
            jnp.dot(mean, wl_ref[...], preferred_element_type=jnp.float32)
            + outr_ref[...]
        )

    return pl.pallas_call(
        tc_body,
        out_shape=jax.ShapeDtypeStruct((n_nodes, hdim), jnp.float32),
    )


def kernel(x, edge_index, emb_weight, W_l, b_l, W_r):
    del x
    n_nodes, dim = emb_weight.shape
    n_edges = edge_index.shape[1]
    hdim = W_l.shape[1]

    ei_flat = edge_index.reshape(2 * n_edges)

    sc_agg = _make_sc_aggregate(n_nodes, n_edges, dim)
    agg_p, deg_p = sc_agg(ei_flat, emb_weight)
    deg2 = deg_p.reshape(NW, n_nodes)

    out_r = _make_tc_right(n_nodes, dim, hdim, blk=2000)(
        emb_weight, W_r, b_l.reshape(1, hdim))
    tc_combine = _make_tc_combine(n_nodes, dim, hdim)
    return tc_combine(agg_p, deg2, out_r, W_l)
